# Initial kernel scaffold; baseline (speedup 1.0000x reference)
import jax
import jax.numpy as jnp
from jax.experimental import pallas as pl


def kernel(x, batch):
    raise NotImplementedError("write your pallas kernel here")


# speedup vs baseline: 10.8055x; 10.8055x over previous
import functools

import jax
import jax.numpy as jnp
from jax import lax
from jax.experimental import pallas as pl
from jax.experimental.pallas import tpu as pltpu
from jax.experimental.pallas import tpu_sc as plsc

N = 100000
D = 128
G = 64
L = 16
NC = 2
NS = 16
NW = NC * NS
CPW = 3128
NPAD = CPW * NW
QPAD = 80
CHUNK = 256
SEGS_PER_W = G // NW

_mesh = plsc.VectorSubcoreMesh(core_axis_name="c", subcore_axis_name="s")


def _worker_id():
    return lax.axis_index("c") * NS + lax.axis_index("s")


def _count_body(batch_hbm, counts_hbm, chunk_v, stage_v):
    w = _worker_id()
    pltpu.sync_copy(batch_hbm.at[pl.ds(w * CPW, CPW)], chunk_v)
    iota = lax.iota(jnp.int32, L)
    for q in range(QPAD // L):
        svec = iota + q * L
        lo = jnp.zeros((L,), jnp.int32)
        hi = jnp.full((L,), CPW, jnp.int32)
        for _ in range(12):
            active = lo < hi
            mid = (lo + hi) // 2
            vals = plsc.load_gather(chunk_v, [jnp.minimum(mid, CPW - 1)])
            less = vals < svec
            lo = jnp.where(jnp.logical_and(active, less), mid + 1, lo)
            hi = jnp.where(jnp.logical_and(active, jnp.logical_not(less)), mid, hi)
        stage_v[pl.ds(q * L, L)] = lo
    pltpu.sync_copy(stage_v, counts_hbm.at[w])


def _pool_body(x_hbm, counts_hbm, out_hbm, cnts_v, buf, stage):
    w = _worker_id()
    pltpu.sync_copy(counts_hbm, cnts_v)
    iota = lax.iota(jnp.int32, L)

    starts = []
    for q in range(QPAD // L):
        acc = jnp.zeros((L,), jnp.int32)
        for ww in range(NW):
            acc = acc + cnts_v[ww, pl.ds(q * L, L)]
        starts.append(acc)

    def extract(e):
        tot = jnp.int32(0)
        for q in range(QPAD // L):
            tot = tot + jnp.sum(jnp.where(iota + q * L == e, starts[q], 0))
        return tot

    for j in range(SEGS_PER_W):
        seg = w * SEGS_PER_W + j
        row_lo = extract(seg)
        row_hi = extract(seg + 1)
        nrows = row_hi - row_lo
        w0 = (row_lo // 8) * 8
        nch = jnp.where(nrows > 0, (row_hi - w0 + CHUNK - 1) // CHUNK, 0)

        def chunk_body(jc, carry):
            wbase = w0 + jc * CHUNK
            dma_base = pl.multiple_of(jnp.minimum(wbase, N - CHUNK), 8)
            shift = wbase - dma_base
            pltpu.sync_copy(x_hbm.at[pl.ds(dma_base, CHUNK)], buf)
            r0 = jnp.maximum(row_lo - wbase, 0)
            r1 = jnp.minimum(row_hi - wbase, CHUNK)

            def row_body(r, c2):
                rr = shift + r
                vs = [buf[rr, pl.ds(k * L, L)] for k in range(D // L)]
                mx = tuple(jnp.maximum(c2[k], vs[k]) for k in range(D // L))
                sm = tuple(c2[D // L + k] + vs[k] for k in range(D // L))
                return mx + sm

            return lax.fori_loop(r0, r1, row_body, carry)

        init = tuple(jnp.full((L,), -jnp.inf, jnp.float32) for _ in range(D // L)) \
            + tuple(jnp.zeros((L,), jnp.float32) for _ in range(D // L))
        res = lax.fori_loop(0, nch, chunk_body, init)

        cnt_vec = jnp.broadcast_to(
            jnp.maximum(nrows, 1).astype(jnp.float32), (L,))
        inv = jnp.ones((L,), jnp.float32) / cnt_vec
        for k in range(D // L):
            stage[pl.ds(k * L, L)] = res[k]
            stage[pl.ds(D + k * L, L)] = res[D // L + k] * inv
        pltpu.sync_copy(stage, out_hbm.at[seg])


_count_kernel = functools.partial(
    pl.kernel,
    out_type=jax.ShapeDtypeStruct((NW, QPAD), jnp.int32),
    mesh=_mesh,
    compiler_params=pltpu.CompilerParams(needs_layout_passes=False),
    scratch_types=[
        pltpu.VMEM((CPW,), jnp.int32),
        pltpu.VMEM((QPAD,), jnp.int32),
    ],
)(_count_body)

_pool_kernel = functools.partial(
    pl.kernel,
    out_type=jax.ShapeDtypeStruct((G, 2 * D), jnp.float32),
    mesh=_mesh,
    compiler_params=pltpu.CompilerParams(needs_layout_passes=False),
    scratch_types=[
        pltpu.VMEM((NW, QPAD), jnp.int32),
        pltpu.VMEM((CHUNK, D), jnp.float32),
        pltpu.VMEM((2 * D,), jnp.float32),
    ],
)(_pool_body)


def kernel(x, batch):
    batch = batch.astype(jnp.int32)
    batch_p = jnp.concatenate(
        [batch, jnp.full((NPAD - N,), jnp.int32(2**30))])
    counts = _count_kernel(batch_p)
    return _pool_kernel(x, counts)


# speedup vs baseline: 13.6563x; 1.2638x over previous
import functools

import jax
import jax.numpy as jnp
from jax import lax
from jax.experimental import pallas as pl
from jax.experimental.pallas import tpu as pltpu
from jax.experimental.pallas import tpu_sc as plsc

N = 100000
D = 128
G = 64
L = 16
NC = 2
NS = 16
NW = NC * NS
CPW = 3128
NPAD = CPW * NW
QPAD = 80
CHUNK = 256
SEGS_PER_W = G // NW

_mesh = plsc.VectorSubcoreMesh(core_axis_name="c", subcore_axis_name="s")


def _worker_id():
    return lax.axis_index("c") * NS + lax.axis_index("s")


def _count_body(batch_hbm, counts_hbm, chunk_v, stage_v):
    w = _worker_id()
    pltpu.sync_copy(batch_hbm.at[pl.ds(w * CPW, CPW)], chunk_v)
    iota = lax.iota(jnp.int32, L)
    for q in range(QPAD // L):
        svec = iota + q * L
        lo = jnp.zeros((L,), jnp.int32)
        hi = jnp.full((L,), CPW, jnp.int32)
        for _ in range(12):
            active = lo < hi
            mid = (lo + hi) // 2
            vals = plsc.load_gather(chunk_v, [jnp.minimum(mid, CPW - 1)])
            less = vals < svec
            lo = jnp.where(jnp.logical_and(active, less), mid + 1, lo)
            hi = jnp.where(jnp.logical_and(active, jnp.logical_not(less)), mid, hi)
        stage_v[pl.ds(q * L, L)] = lo
    pltpu.sync_copy(stage_v, counts_hbm.at[w])


def _pool_body(x_hbm, counts_hbm, out_hbm, cnts_v, buf0, buf1, stage, sem0, sem1):
    w = _worker_id()
    pltpu.sync_copy(counts_hbm, cnts_v)
    iota = lax.iota(jnp.int32, L)

    starts = []
    for q in range(QPAD // L):
        acc = jnp.zeros((L,), jnp.int32)
        for ww in range(NW):
            acc = acc + cnts_v[ww, pl.ds(q * L, L)]
        starts.append(acc)

    def extract(e):
        tot = jnp.int32(0)
        for q in range(QPAD // L):
            tot = tot + jnp.sum(jnp.where(iota + q * L == e, starts[q], 0))
        return tot

    for j in range(SEGS_PER_W):
        seg = w * SEGS_PER_W + j
        row_lo = extract(seg)
        row_hi = extract(seg + 1)
        nrows = row_hi - row_lo
        w0 = (row_lo // 8) * 8
        nch = jnp.where(nrows > 0, (row_hi - w0 + CHUNK - 1) // CHUNK, 0)

        def dma_slice(c):
            return x_hbm.at[
                pl.ds(pl.multiple_of(jnp.minimum(w0 + c * CHUNK, N - CHUNK), 8),
                      CHUNK)]

        def start_copy(c, buf, sem):
            pltpu.async_copy(dma_slice(c), buf, sem)

        def wait_copy(c, buf, sem):
            pltpu.make_async_copy(dma_slice(c), buf, sem).wait()

        def process(c, buf, carry):
            wbase = w0 + c * CHUNK
            dma_base = pl.multiple_of(jnp.minimum(wbase, N - CHUNK), 8)
            shift = wbase - dma_base
            r0 = jnp.maximum(row_lo - wbase, 0)
            r1 = jnp.minimum(row_hi - wbase, CHUNK)

            def row_body(r, c2):
                rr = shift + r
                vs = [buf[rr, pl.ds(k * L, L)] for k in range(D // L)]
                mx = tuple(jnp.maximum(c2[k], vs[k]) for k in range(D // L))
                sm = tuple(c2[D // L + k] + vs[k] for k in range(D // L))
                return mx + sm

            return lax.fori_loop(r0, r1, row_body, carry)

        @pl.when(nch > 0)
        def _():
            start_copy(0, buf0, sem0)

        def pair_body(jp, carry):
            c0 = 2 * jp
            c1 = c0 + 1

            def with_c1(cr):
                start_copy(c1, buf1, sem1)
                return cr

            carry = lax.cond(c1 < nch, with_c1, lambda cr: cr, carry)
            wait_copy(c0, buf0, sem0)
            carry = process(c0, buf0, carry)

            def with_c1_tail(cr):
                def issue_next(cr2):
                    start_copy(c1 + 1, buf0, sem0)
                    return cr2

                cr = lax.cond(c1 + 1 < nch, issue_next, lambda cr2: cr2, cr)
                wait_copy(c1, buf1, sem1)
                return process(c1, buf1, cr)

            return lax.cond(c1 < nch, with_c1_tail, lambda cr: cr, carry)

        init = tuple(jnp.full((L,), -jnp.inf, jnp.float32) for _ in range(D // L)) \
            + tuple(jnp.zeros((L,), jnp.float32) for _ in range(D // L))
        res = lax.fori_loop(0, (nch + 1) // 2, pair_body, init)

        cnt_vec = jnp.broadcast_to(
            jnp.maximum(nrows, 1).astype(jnp.float32), (L,))
        inv = jnp.ones((L,), jnp.float32) / cnt_vec
        for k in range(D // L):
            stage[pl.ds(k * L, L)] = res[k]
            stage[pl.ds(D + k * L, L)] = res[D // L + k] * inv
        pltpu.sync_copy(stage, out_hbm.at[seg])


_count_kernel = functools.partial(
    pl.kernel,
    out_type=jax.ShapeDtypeStruct((NW, QPAD), jnp.int32),
    mesh=_mesh,
    compiler_params=pltpu.CompilerParams(needs_layout_passes=False),
    scratch_types=[
        pltpu.VMEM((CPW,), jnp.int32),
        pltpu.VMEM((QPAD,), jnp.int32),
    ],
)(_count_body)

_pool_kernel = functools.partial(
    pl.kernel,
    out_type=jax.ShapeDtypeStruct((G, 2 * D), jnp.float32),
    mesh=_mesh,
    compiler_params=pltpu.CompilerParams(needs_layout_passes=False),
    scratch_types=[
        pltpu.VMEM((NW, QPAD), jnp.int32),
        pltpu.VMEM((CHUNK, D), jnp.float32),
        pltpu.VMEM((CHUNK, D), jnp.float32),
        pltpu.VMEM((2 * D,), jnp.float32),
        pltpu.SemaphoreType.DMA,
        pltpu.SemaphoreType.DMA,
    ],
)(_pool_body)


def kernel(x, batch):
    batch = batch.astype(jnp.int32)
    batch_p = jnp.concatenate(
        [batch, jnp.full((NPAD - N,), jnp.int32(2**30))])
    counts = _count_kernel(batch_p)
    return _pool_kernel(x, counts)


# speedup vs baseline: 14.8856x; 1.0900x over previous
import functools

import jax
import jax.numpy as jnp
from jax import lax
from jax.experimental import pallas as pl
from jax.experimental.pallas import tpu as pltpu
from jax.experimental.pallas import tpu_sc as plsc

N = 100000
D = 128
G = 64
L = 16
NC = 2
NS = 16
NW = NC * NS
SUB = 16
NPAD = 100096
NSUB = NPAD // SUB
CHUNK = 256
SEGS_PER_W = G // NW

_mesh = plsc.VectorSubcoreMesh(core_axis_name="c", subcore_axis_name="s")


def _pool_body(x_hbm, batch_hbm, bsub_hbm, out_hbm,
               bsub_v, win_v, buf0, buf1, stage, semw, sem0, sem1):
    w = lax.axis_index("c") * NS + lax.axis_index("s")
    iota = lax.iota(jnp.int32, L)

    pltpu.sync_copy(bsub_hbm, bsub_v)
    svec = jnp.minimum(2 * w + iota, G)
    lo = jnp.zeros((L,), jnp.int32)
    hi = jnp.full((L,), NSUB, jnp.int32)
    for _ in range(13):
        active = lo < hi
        mid = (lo + hi) // 2
        vals = plsc.load_gather(bsub_v, [jnp.minimum(mid, NSUB - 1)])
        less = vals < svec
        lo = jnp.where(jnp.logical_and(active, less), mid + 1, lo)
        hi = jnp.where(jnp.logical_and(active, jnp.logical_not(less)), mid, hi)

    wbs = []
    for j in range(SEGS_PER_W + 1):
        p = jnp.sum(jnp.where(iota == j, lo, 0))
        wb = SUB * jnp.maximum(p - 1, 0)
        pltpu.async_copy(batch_hbm.at[pl.ds(pl.multiple_of(wb, 8), SUB)],
                         win_v.at[j], semw)
        wbs.append(wb)
    bounds = []
    for j in range(SEGS_PER_W + 1):
        pltpu.make_async_copy(batch_hbm.at[pl.ds(0, SUB)], win_v.at[j],
                              semw).wait()
    for j in range(SEGS_PER_W + 1):
        s_j = 2 * w + j
        in_win = jnp.sum(jnp.where(win_v[j] < s_j, 1, 0))
        bounds.append(wbs[j] + in_win)

    for j in range(SEGS_PER_W):
        seg = w * SEGS_PER_W + j
        row_lo = bounds[j]
        row_hi = bounds[j + 1]
        nrows = row_hi - row_lo
        w0 = (row_lo // 8) * 8
        nch = jnp.where(nrows > 0, (row_hi - w0 + CHUNK - 1) // CHUNK, 0)

        def dma_slice(c):
            return x_hbm.at[
                pl.ds(pl.multiple_of(jnp.minimum(w0 + c * CHUNK, N - CHUNK), 8),
                      CHUNK)]

        def start_copy(c, buf, sem):
            pltpu.async_copy(dma_slice(c), buf, sem)

        def wait_copy(c, buf, sem):
            pltpu.make_async_copy(dma_slice(c), buf, sem).wait()

        def process(c, buf, carry):
            wbase = w0 + c * CHUNK
            dma_base = pl.multiple_of(jnp.minimum(wbase, N - CHUNK), 8)
            shift = wbase - dma_base
            r0 = jnp.maximum(row_lo - wbase, 0)
            r1 = jnp.minimum(row_hi - wbase, CHUNK)

            def row_body(r, c2):
                rr = shift + r
                vs = [buf[rr, pl.ds(k * L, L)] for k in range(D // L)]
                mx = tuple(jnp.maximum(c2[k], vs[k]) for k in range(D // L))
                sm = tuple(c2[D // L + k] + vs[k] for k in range(D // L))
                return mx + sm

            return lax.fori_loop(r0, r1, row_body, carry)

        @pl.when(nch > 0)
        def _():
            start_copy(0, buf0, sem0)

        def pair_body(jp, carry):
            c0 = 2 * jp
            c1 = c0 + 1

            def with_c1(cr):
                start_copy(c1, buf1, sem1)
                return cr

            carry = lax.cond(c1 < nch, with_c1, lambda cr: cr, carry)
            wait_copy(c0, buf0, sem0)
            carry = process(c0, buf0, carry)

            def with_c1_tail(cr):
                def issue_next(cr2):
                    start_copy(c1 + 1, buf0, sem0)
                    return cr2

                cr = lax.cond(c1 + 1 < nch, issue_next, lambda cr2: cr2, cr)
                wait_copy(c1, buf1, sem1)
                return process(c1, buf1, cr)

            return lax.cond(c1 < nch, with_c1_tail, lambda cr: cr, carry)

        init = tuple(jnp.full((L,), -jnp.inf, jnp.float32) for _ in range(D // L)) \
            + tuple(jnp.zeros((L,), jnp.float32) for _ in range(D // L))
        res = lax.fori_loop(0, (nch + 1) // 2, pair_body, init)

        cnt_vec = jnp.broadcast_to(
            jnp.maximum(nrows, 1).astype(jnp.float32), (L,))
        inv = jnp.ones((L,), jnp.float32) / cnt_vec
        for k in range(D // L):
            stage[pl.ds(k * L, L)] = res[k]
            stage[pl.ds(D + k * L, L)] = res[D // L + k] * inv
        pltpu.sync_copy(stage, out_hbm.at[seg])


_pool_kernel = functools.partial(
    pl.kernel,
    out_type=jax.ShapeDtypeStruct((G, 2 * D), jnp.float32),
    mesh=_mesh,
    compiler_params=pltpu.CompilerParams(needs_layout_passes=False),
    scratch_types=[
        pltpu.VMEM((NSUB,), jnp.int32),
        pltpu.VMEM((SEGS_PER_W + 1, SUB), jnp.int32),
        pltpu.VMEM((CHUNK, D), jnp.float32),
        pltpu.VMEM((CHUNK, D), jnp.float32),
        pltpu.VMEM((2 * D,), jnp.float32),
        pltpu.SemaphoreType.DMA,
        pltpu.SemaphoreType.DMA,
        pltpu.SemaphoreType.DMA,
    ],
)(_pool_body)


def kernel(x, batch):
    batch = batch.astype(jnp.int32)
    batch_p = jnp.concatenate(
        [batch, jnp.full((NPAD - N,), jnp.int32(2**30))])
    return _pool_kernel(x, batch_p, batch_p[::SUB])


# speedup vs baseline: 14.9092x; 1.0016x over previous
import functools

import jax
import jax.numpy as jnp
from jax import lax
from jax.experimental import pallas as pl
from jax.experimental.pallas import tpu as pltpu
from jax.experimental.pallas import tpu_sc as plsc

N = 100000
D = 128
G = 64
L = 16
NC = 2
NS = 16
NW = NC * NS
SUB = 16
NPAD = 100096
NSUB = NPAD // SUB
CHUNK = 256
SEGS_PER_W = G // NW

_mesh = plsc.VectorSubcoreMesh(core_axis_name="c", subcore_axis_name="s")


def _pool_body(x_hbm, batch_hbm, bsub_hbm, out_hbm,
               bsub_v, win_v, buf0, buf1, stage, semw, sem0, sem1):
    w = lax.axis_index("c") * NS + lax.axis_index("s")
    iota = lax.iota(jnp.int32, L)

    pltpu.sync_copy(bsub_hbm, bsub_v)
    svec = jnp.minimum(2 * w + iota, G)
    lo = jnp.zeros((L,), jnp.int32)
    hi = jnp.full((L,), NSUB, jnp.int32)
    for _ in range(13):
        active = lo < hi
        mid = (lo + hi) // 2
        vals = plsc.load_gather(bsub_v, [jnp.minimum(mid, NSUB - 1)])
        less = vals < svec
        lo = jnp.where(jnp.logical_and(active, less), mid + 1, lo)
        hi = jnp.where(jnp.logical_and(active, jnp.logical_not(less)), mid, hi)

    wbs = []
    for j in range(SEGS_PER_W + 1):
        p = jnp.sum(jnp.where(iota == j, lo, 0))
        wb = SUB * jnp.maximum(p - 1, 0)
        pltpu.async_copy(batch_hbm.at[pl.ds(pl.multiple_of(wb, 8), SUB)],
                         win_v.at[j], semw)
        wbs.append(wb)
    bounds = []
    for j in range(SEGS_PER_W + 1):
        pltpu.make_async_copy(batch_hbm.at[pl.ds(0, SUB)], win_v.at[j],
                              semw).wait()
    for j in range(SEGS_PER_W + 1):
        s_j = 2 * w + j
        in_win = jnp.sum(jnp.where(win_v[j] < s_j, 1, 0))
        bounds.append(wbs[j] + in_win)

    for j in range(SEGS_PER_W):
        seg = w * SEGS_PER_W + j
        row_lo = bounds[j]
        row_hi = bounds[j + 1]
        nrows = row_hi - row_lo
        w0 = (row_lo // 8) * 8
        nch = jnp.where(nrows > 0, (row_hi - w0 + CHUNK - 1) // CHUNK, 0)

        def dma_slice(c):
            return x_hbm.at[
                pl.ds(pl.multiple_of(jnp.minimum(w0 + c * CHUNK, N - CHUNK), 8),
                      CHUNK)]

        def start_copy(c, buf, sem):
            pltpu.async_copy(dma_slice(c), buf, sem)

        def wait_copy(c, buf, sem):
            pltpu.make_async_copy(dma_slice(c), buf, sem).wait()

        def process(c, buf, carry):
            wbase = w0 + c * CHUNK
            dma_base = pl.multiple_of(jnp.minimum(wbase, N - CHUNK), 8)
            shift = wbase - dma_base
            r0 = jnp.maximum(row_lo - wbase, 0)
            r1 = jnp.minimum(row_hi - wbase, CHUNK)

            def accum(rr, c2):
                vs = [buf[rr, pl.ds(k * L, L)] for k in range(D // L)]
                mx = tuple(jnp.maximum(c2[k], vs[k]) for k in range(D // L))
                sm = tuple(c2[D // L + k] + vs[k] for k in range(D // L))
                return mx + sm

            def pair_rows(i, c2):
                rr = shift + r0 + 2 * i
                return accum(rr + 1, accum(rr, c2))

            nr = r1 - r0
            carry = lax.fori_loop(0, nr // 2, pair_rows, carry)
            return lax.cond(
                nr % 2 == 1,
                lambda c2: accum(shift + r1 - 1, c2),
                lambda c2: c2, carry)

        @pl.when(nch > 0)
        def _():
            start_copy(0, buf0, sem0)

        def pair_body(jp, carry):
            c0 = 2 * jp
            c1 = c0 + 1

            def with_c1(cr):
                start_copy(c1, buf1, sem1)
                return cr

            carry = lax.cond(c1 < nch, with_c1, lambda cr: cr, carry)
            wait_copy(c0, buf0, sem0)
            carry = process(c0, buf0, carry)

            def with_c1_tail(cr):
                def issue_next(cr2):
                    start_copy(c1 + 1, buf0, sem0)
                    return cr2

                cr = lax.cond(c1 + 1 < nch, issue_next, lambda cr2: cr2, cr)
                wait_copy(c1, buf1, sem1)
                return process(c1, buf1, cr)

            return lax.cond(c1 < nch, with_c1_tail, lambda cr: cr, carry)

        init = tuple(jnp.full((L,), -jnp.inf, jnp.float32) for _ in range(D // L)) \
            + tuple(jnp.zeros((L,), jnp.float32) for _ in range(D // L))
        res = lax.fori_loop(0, (nch + 1) // 2, pair_body, init)

        cnt_vec = jnp.broadcast_to(
            jnp.maximum(nrows, 1).astype(jnp.float32), (L,))
        inv = jnp.ones((L,), jnp.float32) / cnt_vec
        for k in range(D // L):
            stage[pl.ds(k * L, L)] = res[k]
            stage[pl.ds(D + k * L, L)] = res[D // L + k] * inv
        pltpu.sync_copy(stage, out_hbm.at[seg])


_pool_kernel = functools.partial(
    pl.kernel,
    out_type=jax.ShapeDtypeStruct((G, 2 * D), jnp.float32),
    mesh=_mesh,
    compiler_params=pltpu.CompilerParams(needs_layout_passes=False),
    scratch_types=[
        pltpu.VMEM((NSUB,), jnp.int32),
        pltpu.VMEM((SEGS_PER_W + 1, SUB), jnp.int32),
        pltpu.VMEM((CHUNK, D), jnp.float32),
        pltpu.VMEM((CHUNK, D), jnp.float32),
        pltpu.VMEM((2 * D,), jnp.float32),
        pltpu.SemaphoreType.DMA,
        pltpu.SemaphoreType.DMA,
        pltpu.SemaphoreType.DMA,
    ],
)(_pool_body)


def kernel(x, batch):
    batch = batch.astype(jnp.int32)
    bsub = jnp.concatenate(
        [batch[::SUB], jnp.full((NSUB - N // SUB,), jnp.int32(2**30))])
    return _pool_kernel(x, batch, bsub)
